# Initial kernel scaffold; baseline (speedup 1.0000x reference)
import jax
import jax.numpy as jnp
from jax.experimental import pallas as pl


def kernel(x_o, x_a, features1, edge_index, edge_type, edge_type1, idx, W1, root1, b1, W2, root2, b2, attt, Wb, bbias, Wc, bc):
    raise NotImplementedError("write your pallas kernel here")


# speedup vs baseline: 1.9217x; 1.9217x over previous
import functools

import jax
import jax.numpy as jnp
from jax import lax
from jax.experimental import pallas as pl
from jax.experimental.pallas import tpu as pltpu
from jax.experimental.pallas import tpu_sc as plsc

N = 10000
E = 320000
R = 65
F_IN = 128
H1 = 64
H2 = 32
B = 4096

RN = R * N
TPAD = 650240
NC = 2
NS = 16
NW = NC * NS
EPW = E // NW
KB = 80
NBLK = EPW // KB
CNT_SL = TPAD // NS
NPAD = 10240
N_SL = NPAD // NS
BPW = B // NW
CNT_CH = 8128

@functools.cache
def _mesh():
    return plsc.VectorSubcoreMesh(core_axis_name="c", subcore_axis_name="s",
                                  num_cores=NC, num_subcores=NS)


def _wid():
    return lax.axis_index("s") * NC + lax.axis_index("c")


def _sc_counts(seg0_h, seg1_h, ones_h, zeros_h, out_h,
               cnt0_sh, cnt1_sh, ones_v, idx_v, zb_v):
    cid = lax.axis_index("c")
    sid = lax.axis_index("s")
    wid = _wid()
    z0 = sid * CNT_SL
    pltpu.sync_copy(zeros_h, zb_v)
    for q in range(CNT_SL // CNT_CH):
        pltpu.sync_copy(zb_v, cnt0_sh.at[pl.ds(z0 + q * CNT_CH, CNT_CH)])
        pltpu.sync_copy(zb_v, cnt1_sh.at[pl.ds(z0 + q * CNT_CH, CNT_CH)])
    pltpu.sync_copy(ones_h, ones_v)
    plsc.subcore_barrier()
    base = wid * EPW

    def blk(j, carry):
        off = base + j * KB
        pltpu.sync_copy(seg0_h.at[pl.ds(off, KB)], idx_v)
        pltpu.sync_copy(ones_v, cnt0_sh.at[idx_v], add=True)
        pltpu.sync_copy(seg1_h.at[pl.ds(off, KB)], idx_v)
        pltpu.sync_copy(ones_v, cnt1_sh.at[idx_v], add=True)
        return carry

    lax.fori_loop(0, NBLK, blk, 0)
    plsc.subcore_barrier()
    for t, sh in ((0, cnt0_sh), (1, cnt1_sh)):
        for q in range(CNT_SL // CNT_CH):
            pltpu.sync_copy(sh.at[pl.ds(z0 + q * CNT_CH, CNT_CH)], zb_v)
            pltpu.sync_copy(
                zb_v,
                out_h.at[pl.ds((cid * 2 + t) * TPAD + z0 + q * CNT_CH, CNT_CH)])


@functools.cache
def _counts_kernel():
    return pl.kernel(
        _sc_counts,
        out_type=jax.ShapeDtypeStruct((NC * 2 * TPAD,), jnp.float32),
        mesh=_mesh(),
        scratch_types=[
            pltpu.VMEM_SHARED((TPAD,), jnp.float32),
            pltpu.VMEM_SHARED((TPAD,), jnp.float32),
            pltpu.VMEM((KB,), jnp.float32),
            pltpu.VMEM((KB,), jnp.int32),
            pltpu.VMEM((CNT_CH,), jnp.float32),
        ],
    )


def _counts_call(*args):
    return _counts_kernel()(*args)


def _scale_rows(msg_ref, ew_ref, h):
    nh = h // 16

    def grp(g, carry):
        w16 = ew_ref[pl.ds(g * 16, 16)]
        for e in range(16):
            ei = g * 16 + e
            w = w16[e]
            for k in range(nh):
                sl = pl.ds(k * 16, 16)
                msg_ref[ei, sl] = msg_ref[ei, sl] * w
        return carry

    lax.fori_loop(0, KB // 16, grp, 0)


ZROWS = 128


def _make_agg(h, n_tables, branches):
    nb = len(branches)
    any_main = any(not alt for _, alt in branches)
    any_alt = any(alt for _, alt in branches)

    def body(*refs):
        tabs = refs[:n_tables]
        it = iter(refs[n_tables:])
        gidx_h, gidx1_h, dst_h, seg0_h, seg1_h = (next(it) for _ in range(5))
        inv0_h, inv1_h, zeros_h, out_h = (next(it) for _ in range(4))
        accs = [next(it) for _ in range(nb)]
        msgs = [next(it) for _ in range(nb)]
        g_v, g1_v, dst_v, s0_v, s1_v, ew0_v, ew1_v, zb_v = (
            next(it) for _ in range(8))
        sems = [next(it) for _ in range(nb)]
        seme0, seme1 = next(it), next(it)
        cid = lax.axis_index("c")
        sid = lax.axis_index("s")
        wid = _wid()
        r0 = sid * N_SL
        pltpu.sync_copy(zeros_h, zb_v)
        for acc in accs:
            for q in range(N_SL // ZROWS):
                pltpu.sync_copy(zb_v, acc.at[pl.ds(r0 + q * ZROWS, ZROWS)])
        plsc.subcore_barrier()
        base = wid * EPW
        gsel = [g1_v if alt else g_v for _, alt in branches]
        esel = [ew1_v if alt else ew0_v for _, alt in branches]

        def blk(j, carry):
            off = base + j * KB
            if any_main:
                pltpu.sync_copy(gidx_h.at[pl.ds(off, KB)], g_v)
                pltpu.sync_copy(seg0_h.at[pl.ds(off, KB)], s0_v)
            if any_alt:
                pltpu.sync_copy(gidx1_h.at[pl.ds(off, KB)], g1_v)
                pltpu.sync_copy(seg1_h.at[pl.ds(off, KB)], s1_v)
            pltpu.sync_copy(dst_h.at[pl.ds(off, KB)], dst_v)
            ecps = []
            if any_main:
                ecps.append(pltpu.async_copy(inv0_h.at[s0_v], ew0_v, seme0))
            if any_alt:
                ecps.append(pltpu.async_copy(inv1_h.at[s1_v], ew1_v, seme1))
            cps = []
            for b, (slot, _) in enumerate(branches):
                cps.append(pltpu.async_copy(tabs[slot].at[gsel[b]],
                                            msgs[b], sems[b]))
            for e in ecps:
                e.wait()
            for b in range(nb):
                cps[b].wait()
                _scale_rows(msgs[b], esel[b], h)
                pltpu.sync_copy(msgs[b], accs[b].at[dst_v], add=True)
            return carry

        lax.fori_loop(0, NBLK, blk, 0)
        plsc.subcore_barrier()
        for b in range(nb):
            for q in range(N_SL // ZROWS):
                pltpu.sync_copy(accs[b].at[pl.ds(r0 + q * ZROWS, ZROWS)], zb_v)
                pltpu.sync_copy(
                    zb_v,
                    out_h.at[pl.ds((cid * nb + b) * NPAD + r0 + q * ZROWS,
                                   ZROWS)])

    scratch = (
        [pltpu.VMEM_SHARED((NPAD, h), jnp.float32)] * nb
        + [pltpu.VMEM((KB, h), jnp.float32)] * nb
        + [pltpu.VMEM((KB,), jnp.int32)] * 5
        + [pltpu.VMEM((KB,), jnp.float32)] * 2
        + [pltpu.VMEM((ZROWS, h), jnp.float32)]
        + [pltpu.SemaphoreType.DMA] * (nb + 2)
    )
    return pl.kernel(
        body,
        out_type=jax.ShapeDtypeStruct((NC * nb * NPAD, h), jnp.float32),
        mesh=_mesh(),
        compiler_params=pltpu.CompilerParams(use_tc_tiling_on_sc=False),
        scratch_types=scratch,
    )


_make_agg = functools.cache(_make_agg)


def _agg_l1a(*args):
    return _make_agg(H1, 2, ((0, False), (1, False)))(*args)


def _agg_l1b(*args):
    return _make_agg(H1, 1, ((0, True),))(*args)


def _agg_l2(*args):
    return _make_agg(H2, 3, ((0, False), (1, False), (2, True)))(*args)


def _sc_pair_gather(q_h, r_h, aa_h, bb_h, out_h, i_v, rows_v, sem):
    wid = _wid()
    base = wid * BPW
    pltpu.sync_copy(aa_h.at[pl.ds(base, BPW)], i_v)
    pltpu.async_copy(q_h.at[i_v], rows_v, sem).wait()
    pltpu.sync_copy(rows_v, out_h.at[0, pl.ds(base, BPW)])
    pltpu.sync_copy(bb_h.at[pl.ds(base, BPW)], i_v)
    pltpu.async_copy(r_h.at[i_v], rows_v, sem).wait()
    pltpu.sync_copy(rows_v, out_h.at[1, pl.ds(base, BPW)])


@functools.cache
def _pair_gather_kernel():
    return pl.kernel(
        _sc_pair_gather,
        out_type=jax.ShapeDtypeStruct((2, B, 128), jnp.float32),
        mesh=_mesh(),
        scratch_types=[
            pltpu.VMEM((BPW,), jnp.int32),
            pltpu.VMEM((BPW, 128), jnp.float32),
            pltpu.SemaphoreType.DMA,
        ],
    )


def _pair_gather_call(*args):
    return _pair_gather_kernel()(*args)


def _relmm_body(x_ref, w_ref, o_ref):
    o_ref[0] = jnp.dot(x_ref[...], w_ref[0], preferred_element_type=jnp.float32)


def _rel_matmul(x, w):
    n, f = x.shape
    r, _, h = w.shape
    bn = 2000
    out = pl.pallas_call(
        _relmm_body,
        grid=(n // bn, r),
        in_specs=[
            pl.BlockSpec((bn, f), lambda i, j: (i, 0)),
            pl.BlockSpec((1, f, h), lambda i, j: (j, 0, 0)),
        ],
        out_specs=pl.BlockSpec((1, bn, h), lambda i, j: (j, i, 0)),
        out_shape=jax.ShapeDtypeStruct((r, n, h), jnp.float32),
    )(x, w)
    return out.reshape(r * n, h)


def _inv_body(c_ref, o_ref):
    s = c_ref[0] + c_ref[1]
    o_ref[...] = 1.0 / jnp.maximum(s, 1.0)


def _inv_counts(counts):
    c4 = counts.reshape(NC, 2, TPAD // 128, 128)
    nb = TPAD // 128
    bn = 1016
    out = pl.pallas_call(
        _inv_body,
        grid=(nb // bn,),
        in_specs=[pl.BlockSpec((NC, 2, bn, 128), lambda i: (0, 0, i, 0))],
        out_specs=pl.BlockSpec((2, bn, 128), lambda i: (0, i, 0)),
        out_shape=jax.ShapeDtypeStruct((2, nb, 128), jnp.float32),
    )(c4)
    return out.reshape(2, TPAD)


def _combine1_body(acc_ref, xo_ref, xa_ref, rt_ref, b_ref, o_ref):
    ro = jnp.dot(xo_ref[...], rt_ref[...], preferred_element_type=jnp.float32)
    ra = jnp.dot(xa_ref[...], rt_ref[...], preferred_element_type=jnp.float32)
    s = acc_ref[0] + acc_ref[1]
    bv = b_ref[0]
    o_ref[0] = jax.nn.relu(s[0] + ro + bv)
    o_ref[1] = jax.nn.relu(s[1] + ra + bv)
    o_ref[2] = jax.nn.relu(s[2] + ro + bv)


def _combine1(acc, x_o, x_a, root, bias):
    bn = 2000
    return pl.pallas_call(
        _combine1_body,
        grid=(N // bn,),
        in_specs=[
            pl.BlockSpec((NC, 3, bn, H1), lambda i: (0, 0, i, 0)),
            pl.BlockSpec((bn, F_IN), lambda i: (i, 0)),
            pl.BlockSpec((bn, F_IN), lambda i: (i, 0)),
            pl.BlockSpec((F_IN, H1), lambda i: (0, 0)),
            pl.BlockSpec((1, H1), lambda i: (0, 0)),
        ],
        out_specs=pl.BlockSpec((3, bn, H1), lambda i: (0, i, 0)),
        out_shape=jax.ShapeDtypeStruct((3, N, H1), jnp.float32),
    )(acc, x_o, x_a, root, bias.reshape(1, H1))


def _combine2_body(acc_ref, x1_ref, rt_ref, b_ref, o_ref):
    s = acc_ref[0] + acc_ref[1]
    bv = b_ref[0]
    for b in range(3):
        rb = jnp.dot(x1_ref[b], rt_ref[...], preferred_element_type=jnp.float32)
        o_ref[b] = s[b] + rb + bv


def _combine2(acc, x1, root, bias):
    bn = 2000
    return pl.pallas_call(
        _combine2_body,
        grid=(N // bn,),
        in_specs=[
            pl.BlockSpec((NC, 3, bn, H2), lambda i: (0, 0, i, 0)),
            pl.BlockSpec((3, bn, H1), lambda i: (0, i, 0)),
            pl.BlockSpec((H1, H2), lambda i: (0, 0)),
            pl.BlockSpec((1, H2), lambda i: (0, 0)),
        ],
        out_specs=pl.BlockSpec((3, bn, H2), lambda i: (0, i, 0)),
        out_shape=jax.ShapeDtypeStruct((3, N, H2), jnp.float32),
    )(acc, x1, root, bias.reshape(1, H2))


def _postu_body(x2_ref, wb_ref, o_ref):
    h = jax.nn.sigmoid(jnp.mean(x2_ref[...], axis=0))
    u = jnp.dot(wb_ref[0], h[:, None], preferred_element_type=jnp.float32)
    col = lax.broadcasted_iota(jnp.int32, (H2, 128), 1)
    o_ref[...] = jnp.where(col == 0, u, 0.0)


def _post_u(x2_o, wb):
    return pl.pallas_call(
        _postu_body,
        in_specs=[
            pl.BlockSpec((N, H2), lambda: (0, 0)),
            pl.BlockSpec((1, H2, H2), lambda: (0, 0, 0)),
        ],
        out_specs=pl.BlockSpec((H2, 128), lambda: (0, 0)),
        out_shape=jax.ShapeDtypeStruct((H2, 128), jnp.float32),
    )(x2_o, wb)


def _postb_body(x1o_ref, x2_ref, f1_ref, u_ref, att_ref, wq1_ref, wq2_ref,
                wr1_ref, wr2_ref, bb_ref, ros_ref, rosa_ref, q_ref, r_ref):
    bb = bb_ref[0, 0]
    bn = x2_ref.shape[1]
    x2f = x2_ref[...].reshape(3 * bn, H2)
    p = jnp.dot(x2f, u_ref[...], preferred_element_type=jnp.float32)
    bil = (p[:, :1] + bb).reshape(3, bn, 1)
    ros_ref[...] = jnp.concatenate([bil[0], bil[1]], axis=1)
    rosa_ref[...] = jnp.concatenate([bil[0], bil[2]], axis=1)
    fin = jnp.concatenate([att_ref[0, 0] * x1o_ref[...],
                           att_ref[0, 1] * x2_ref[0]], axis=1)
    f1 = f1_ref[...]
    q_ref[...] = (jnp.dot(fin, wq1_ref[...], preferred_element_type=jnp.float32)
                  + jnp.dot(f1, wq2_ref[...], preferred_element_type=jnp.float32))
    r_ref[...] = (jnp.dot(fin, wr1_ref[...], preferred_element_type=jnp.float32)
                  + jnp.dot(f1, wr2_ref[...], preferred_element_type=jnp.float32))


def _post_b(x1_o, x2, features1, u, attt, wq1, wq2, wr1, wr2, bbias):
    bn = 2000
    return pl.pallas_call(
        _postb_body,
        grid=(N // bn,),
        in_specs=[
            pl.BlockSpec((bn, H1), lambda i: (i, 0)),
            pl.BlockSpec((3, bn, H2), lambda i: (0, i, 0)),
            pl.BlockSpec((bn, F_IN), lambda i: (i, 0)),
            pl.BlockSpec((H2, 128), lambda i: (0, 0)),
            pl.BlockSpec((1, 2), lambda i: (0, 0)),
            pl.BlockSpec((H1 + H2, 128), lambda i: (0, 0)),
            pl.BlockSpec((F_IN, 128), lambda i: (0, 0)),
            pl.BlockSpec((H1 + H2, 128), lambda i: (0, 0)),
            pl.BlockSpec((F_IN, 128), lambda i: (0, 0)),
            pl.BlockSpec((1, 1), lambda i: (0, 0)),
        ],
        out_specs=[
            pl.BlockSpec((bn, 2), lambda i: (i, 0)),
            pl.BlockSpec((bn, 2), lambda i: (i, 0)),
            pl.BlockSpec((bn, 128), lambda i: (i, 0)),
            pl.BlockSpec((bn, 128), lambda i: (i, 0)),
        ],
        out_shape=[
            jax.ShapeDtypeStruct((N, 2), jnp.float32),
            jax.ShapeDtypeStruct((N, 2), jnp.float32),
            jax.ShapeDtypeStruct((N, 128), jnp.float32),
            jax.ShapeDtypeStruct((N, 128), jnp.float32),
        ],
    )(x1_o, x2, features1, u, attt, wq1, wq2, wr1, wr2, bbias)


def _logadd_body(g_ref, bc_ref, o_ref):
    s = g_ref[0] + g_ref[1] + bc_ref[0]
    o_ref[...] = s[:, :R]


def _logadd(gathered, bc_pad):
    return pl.pallas_call(
        _logadd_body,
        in_specs=[
            pl.BlockSpec((2, B, 128), lambda: (0, 0, 0)),
            pl.BlockSpec((1, 128), lambda: (0, 0)),
        ],
        out_specs=pl.BlockSpec((B, R), lambda: (0, 0)),
        out_shape=jax.ShapeDtypeStruct((B, R), jnp.float32),
    )(gathered, bc_pad)


def kernel(x_o, x_a, features1, edge_index, edge_type, edge_type1, idx,
           W1, root1, b1, W2, root2, b2, attt, Wb, bbias, Wc, bc):
    src = edge_index[0]
    dst = edge_index[1]
    n32 = jnp.int32(N)
    gidx = edge_type * n32 + src
    gidx1 = edge_type1 * n32 + src
    seg0 = edge_type * n32 + dst
    seg1 = edge_type1 * n32 + dst

    ones_kb = jnp.ones((KB,), jnp.float32)
    zeros_tp = jnp.zeros((CNT_CH,), jnp.float32)
    zeros_h1 = jnp.zeros((ZROWS, H1), jnp.float32)
    zeros_h2 = jnp.zeros((ZROWS, H2), jnp.float32)

    counts = _counts_call(seg0, seg1, ones_kb, zeros_tp)
    inv = _inv_counts(counts.reshape(NC, 2, TPAD))
    inv0 = inv[0]
    inv1 = inv[1]

    t1_o = _rel_matmul(x_o, W1)
    t1_a = _rel_matmul(x_a, W1)
    acc1a = _agg_l1a(t1_o, t1_a, gidx, gidx1, dst, seg0, seg1, inv0, inv1,
                     zeros_h1)
    acc1b = _agg_l1b(t1_o, gidx, gidx1, dst, seg0, seg1, inv0, inv1,
                     zeros_h1)
    acc1 = jnp.concatenate([acc1a.reshape(NC, 2, NPAD, H1)[:, :, :N],
                            acc1b.reshape(NC, 1, NPAD, H1)[:, :, :N]], axis=1)
    x1 = _combine1(acc1, x_o, x_a, root1, b1)

    t2_o = _rel_matmul(x1[0], W2)
    t2_a = _rel_matmul(x1[1], W2)
    t2_b = _rel_matmul(x1[2], W2)
    acc2 = _agg_l2(t2_o, t2_a, t2_b, gidx, gidx1, dst, seg0, seg1, inv0, inv1,
                   zeros_h2)
    acc2 = acc2.reshape(NC, 3, NPAD, H2)[:, :, :N]
    x2 = _combine2(acc2, x1, root2, b2)
    x2_o = x2[0]

    u = _post_u(x2_o, Wb)

    wq1 = jnp.zeros((H1 + H2, 128), jnp.float32).at[:, :R].set(Wc[:, :96].T)
    wq2 = jnp.zeros((F_IN, 128), jnp.float32).at[:, :R].set(Wc[:, 96:224].T)
    wr1 = jnp.zeros((H1 + H2, 128), jnp.float32).at[:, :R].set(Wc[:, 224:320].T)
    wr2 = jnp.zeros((F_IN, 128), jnp.float32).at[:, :R].set(Wc[:, 320:448].T)
    bc_pad = jnp.zeros((1, 128), jnp.float32).at[0, :R].set(bc)

    ret_os, ret_os_a, q, rr = _post_b(
        x1[0], x2, features1, u, attt.reshape(1, 2),
        wq1, wq2, wr1, wr2, bbias.reshape(1, 1))

    gathered = _pair_gather_call(q, rr, idx[0], idx[1])
    log = _logadd(gathered, bc_pad)

    return (log, ret_os, ret_os_a, x2_o)


# speedup vs baseline: 3.2342x; 1.6830x over previous
import functools

import jax
import jax.numpy as jnp
from jax import lax
from jax.experimental import pallas as pl
from jax.experimental.pallas import tpu as pltpu
from jax.experimental.pallas import tpu_sc as plsc

N = 10000
E = 320000
R = 65
F_IN = 128
H1 = 64
H2 = 32
B = 4096

RN = R * N
TPAD = 650240
NC = 2
NS = 16
NW = NC * NS
EPW = E // NW
KB = 80
NBLK = EPW // KB
SB = 25
NSB = NBLK // SB
EROWS = E // KB
CNT_SL = TPAD // NS
NPAD = 10240
N_SL = NPAD // NS
BPW = B // NW
CNT_CH = 8128
ZROWS = 64


@functools.cache
def _mesh():
    return plsc.VectorSubcoreMesh(core_axis_name="c", subcore_axis_name="s",
                                  num_cores=NC, num_subcores=NS)


def _wid():
    return lax.axis_index("s") * NC + lax.axis_index("c")


def _sc_counts(seg0_h, seg1_h, ones_h, zeros_h, out_h,
               cnt0_sh, cnt1_sh, ones_v, s0_v, s1_v, zb_v, sem):
    cid = lax.axis_index("c")
    sid = lax.axis_index("s")
    wid = _wid()
    z0 = sid * CNT_SL
    pltpu.sync_copy(zeros_h, zb_v)
    for q in range(CNT_SL // CNT_CH):
        pltpu.sync_copy(zb_v.at[pl.ds(0, CNT_CH)],
                        cnt0_sh.at[pl.ds(z0 + q * CNT_CH, CNT_CH)])
        pltpu.sync_copy(zb_v.at[pl.ds(0, CNT_CH)],
                        cnt1_sh.at[pl.ds(z0 + q * CNT_CH, CNT_CH)])
    pltpu.sync_copy(ones_h, ones_v)
    plsc.subcore_barrier()
    row0 = wid * NBLK
    for s in range(NSB):
        srow = row0 + s * SB
        pltpu.sync_copy(seg0_h.at[pl.ds(srow, SB)], s0_v)
        pltpu.sync_copy(seg1_h.at[pl.ds(srow, SB)], s1_v)

        def blk(jj, carry):
            pltpu.async_copy(ones_v, cnt0_sh.at[s0_v.at[jj]], sem, add=True)
            pltpu.async_copy(ones_v, cnt1_sh.at[s1_v.at[jj]], sem, add=True)
            return carry

        lax.fori_loop(0, SB, blk, 0)

        def drain(jj, carry):
            pltpu.make_async_copy(ones_v, cnt0_sh.at[s0_v.at[jj]], sem).wait()
            pltpu.make_async_copy(ones_v, cnt1_sh.at[s1_v.at[jj]], sem).wait()
            return carry

        lax.fori_loop(0, SB, drain, 0)
    plsc.subcore_barrier()
    for t, sh in ((0, cnt0_sh), (1, cnt1_sh)):
        for q in range(CNT_SL // CNT_CH):
            pltpu.sync_copy(sh.at[pl.ds(z0 + q * CNT_CH, CNT_CH)], zb_v)
            pltpu.sync_copy(
                zb_v,
                out_h.at[pl.ds((cid * 2 + t) * TPAD + z0 + q * CNT_CH, CNT_CH)])


@functools.cache
def _counts_kernel():
    return pl.kernel(
        _sc_counts,
        out_type=jax.ShapeDtypeStruct((NC * 2 * TPAD,), jnp.float32),
        mesh=_mesh(),
        compiler_params=pltpu.CompilerParams(use_tc_tiling_on_sc=False),
        scratch_types=[
            pltpu.VMEM_SHARED((TPAD,), jnp.float32),
            pltpu.VMEM_SHARED((TPAD,), jnp.float32),
            pltpu.VMEM((KB,), jnp.float32),
            pltpu.VMEM((SB, KB), jnp.int32),
            pltpu.VMEM((SB, KB), jnp.int32),
            pltpu.VMEM((CNT_CH,), jnp.float32),
            pltpu.SemaphoreType.DMA,
        ],
    )


def _counts_call(*args):
    return _counts_kernel()(*args)


def _scale_rows(msg_ref, ew_ref, h):
    nh = h // 16

    def grp(g, carry):
        w16 = ew_ref[pl.ds(g * 16, 16)]
        for e in range(16):
            ei = g * 16 + e
            w = w16[e]
            for k in range(nh):
                sl = pl.ds(k * 16, 16)
                msg_ref[ei, sl] = msg_ref[ei, sl] * w
        return carry

    lax.fori_loop(0, KB // 16, grp, 0)


def _make_agg(h, n_tables, branches):
    nb = len(branches)
    any_main = any(not alt for _, alt in branches)
    any_alt = any(alt for _, alt in branches)

    def body(*refs):
        tabs = refs[:n_tables]
        it = iter(refs[n_tables:])
        gidx_h, gidx1_h, dst_h, seg0_h, seg1_h = (next(it) for _ in range(5))
        inv0_h, inv1_h, zeros_h, out_h = (next(it) for _ in range(4))
        accs = [next(it) for _ in range(nb)]
        msgs = [[next(it), next(it)] for _ in range(nb)]
        g_v, g1_v, dst_v, s0_v, s1_v, zb_v = (next(it) for _ in range(6))
        ew0_v = [next(it), next(it)]
        ew1_v = [next(it), next(it)]
        sems = [[next(it), next(it)] for _ in range(nb)]
        seme0 = [next(it), next(it)]
        seme1 = [next(it), next(it)]
        cid = lax.axis_index("c")
        sid = lax.axis_index("s")
        wid = _wid()
        r0 = sid * N_SL
        pltpu.sync_copy(zeros_h, zb_v)
        for acc in accs:
            for q in range(N_SL // ZROWS):
                pltpu.sync_copy(zb_v, acc.at[pl.ds(r0 + q * ZROWS, ZROWS)])
        plsc.subcore_barrier()
        row0 = wid * NBLK
        gsel = [g1_v if alt else g_v for _, alt in branches]
        esel = [ew1_v if alt else ew0_v for _, alt in branches]

        def fire(jj, par):
            if any_main:
                pltpu.async_copy(inv0_h.at[s0_v.at[jj]], ew0_v[par],
                                 seme0[par])
            if any_alt:
                pltpu.async_copy(inv1_h.at[s1_v.at[jj]], ew1_v[par],
                                 seme1[par])
            for b, (slot, _) in enumerate(branches):
                pltpu.async_copy(tabs[slot].at[gsel[b].at[jj]],
                                 msgs[b][par], sems[b][par])

        def process(jj, par):
            if any_main:
                pltpu.make_async_copy(inv0_h.at[s0_v.at[jj]], ew0_v[par],
                                      seme0[par]).wait()
            if any_alt:
                pltpu.make_async_copy(inv1_h.at[s1_v.at[jj]], ew1_v[par],
                                      seme1[par]).wait()
            for b, (slot, _) in enumerate(branches):
                pltpu.make_async_copy(tabs[slot].at[gsel[b].at[jj]],
                                      msgs[b][par], sems[b][par]).wait()
                _scale_rows(msgs[b][par],
                            ew1_v[par] if branches[b][1] else ew0_v[par], h)
                pltpu.sync_copy(msgs[b][par], accs[b].at[dst_v.at[jj]],
                                add=True)

        for s in range(NSB):
            srow = row0 + s * SB
            if any_main:
                pltpu.sync_copy(gidx_h.at[pl.ds(srow, SB)], g_v)
                pltpu.sync_copy(seg0_h.at[pl.ds(srow, SB)], s0_v)
            if any_alt:
                pltpu.sync_copy(gidx1_h.at[pl.ds(srow, SB)], g1_v)
                pltpu.sync_copy(seg1_h.at[pl.ds(srow, SB)], s1_v)
            pltpu.sync_copy(dst_h.at[pl.ds(srow, SB)], dst_v)
            fire(0, 0)

            def pair(t, carry):
                j0 = 2 * t
                fire(j0 + 1, 1)
                process(j0, 0)
                fire(j0 + 2, 0)
                process(j0 + 1, 1)
                return carry

            lax.fori_loop(0, (SB - 1) // 2, pair, 0)
            process(SB - 1, 0)
        plsc.subcore_barrier()
        for b in range(nb):
            for q in range(N_SL // ZROWS):
                pltpu.sync_copy(accs[b].at[pl.ds(r0 + q * ZROWS, ZROWS)], zb_v)
                pltpu.sync_copy(
                    zb_v,
                    out_h.at[pl.ds((cid * nb + b) * NPAD + r0 + q * ZROWS,
                                   ZROWS)])

    scratch = (
        [pltpu.VMEM_SHARED((NPAD, h), jnp.float32)] * nb
        + [pltpu.VMEM((KB, h), jnp.float32)] * (2 * nb)
        + [pltpu.VMEM((SB, KB), jnp.int32)] * 5
        + [pltpu.VMEM((ZROWS, h), jnp.float32)]
        + [pltpu.VMEM((KB,), jnp.float32)] * 4
        + [pltpu.SemaphoreType.DMA] * (2 * nb + 4)
    )
    return pl.kernel(
        body,
        out_type=jax.ShapeDtypeStruct((NC * nb * NPAD, h), jnp.float32),
        mesh=_mesh(),
        compiler_params=pltpu.CompilerParams(use_tc_tiling_on_sc=False),
        scratch_types=scratch,
    )


_make_agg = functools.cache(_make_agg)


def _agg_l1a(*args):
    return _make_agg(H1, 2, ((0, False), (1, False)))(*args)


def _agg_l1b(*args):
    return _make_agg(H1, 1, ((0, True),))(*args)


def _agg_l2(*args):
    return _make_agg(H2, 3, ((0, False), (1, False), (2, True)))(*args)


def _sc_pair_gather(q_h, r_h, aa_h, bb_h, out_h, i_v, rows_v, sem):
    wid = _wid()
    base = wid * BPW
    pltpu.sync_copy(aa_h.at[pl.ds(base, BPW)], i_v)
    pltpu.async_copy(q_h.at[i_v], rows_v, sem).wait()
    pltpu.sync_copy(rows_v, out_h.at[0, pl.ds(base, BPW)])
    pltpu.sync_copy(bb_h.at[pl.ds(base, BPW)], i_v)
    pltpu.async_copy(r_h.at[i_v], rows_v, sem).wait()
    pltpu.sync_copy(rows_v, out_h.at[1, pl.ds(base, BPW)])


@functools.cache
def _pair_gather_kernel():
    return pl.kernel(
        _sc_pair_gather,
        out_type=jax.ShapeDtypeStruct((2, B, 128), jnp.float32),
        mesh=_mesh(),
        scratch_types=[
            pltpu.VMEM((BPW,), jnp.int32),
            pltpu.VMEM((BPW, 128), jnp.float32),
            pltpu.SemaphoreType.DMA,
        ],
    )


def _pair_gather_call(*args):
    return _pair_gather_kernel()(*args)


def _make_relmm_body(m):
    def body(*refs):
        wb = refs[m][0]
        for i in range(m):
            refs[m + 1 + i][0] = jnp.dot(refs[i][...], wb,
                                         preferred_element_type=jnp.float32)
    return body


def _rel_matmul_multi(xs, w_ext):
    m = len(xs)
    n, f = xs[0].shape
    rr, _, h = w_ext.shape
    outs = pl.pallas_call(
        _make_relmm_body(m),
        grid=(rr,),
        in_specs=[pl.BlockSpec((n, f), lambda r: (0, 0)) for _ in range(m)]
        + [pl.BlockSpec((1, f, h), lambda r: (r, 0, 0))],
        out_specs=[pl.BlockSpec((1, n, h), lambda r: (r, 0, 0))
                   for _ in range(m)],
        out_shape=[jax.ShapeDtypeStruct((rr, n, h), jnp.float32)
                   for _ in range(m)],
    )(*xs, w_ext)
    return [o.reshape(rr * n, h) for o in outs]


def _inv_body(c_ref, o_ref):
    s = c_ref[0] + c_ref[1]
    o_ref[...] = 1.0 / jnp.maximum(s, 1.0)


def _inv_counts(counts):
    c4 = counts.reshape(NC, 2, TPAD // 128, 128)
    nb = TPAD // 128
    bn = 1016
    out = pl.pallas_call(
        _inv_body,
        grid=(nb // bn,),
        in_specs=[pl.BlockSpec((NC, 2, bn, 128), lambda i: (0, 0, i, 0))],
        out_specs=pl.BlockSpec((2, bn, 128), lambda i: (0, i, 0)),
        out_shape=jax.ShapeDtypeStruct((2, nb, 128), jnp.float32),
    )(c4)
    return out.reshape(2, TPAD)


def _combine1a_body(acc_ref, ro_ref, ra_ref, b_ref, o_ref):
    s = acc_ref[0] + acc_ref[1]
    bv = b_ref[0]
    o_ref[0] = jax.nn.relu(s[0] + ro_ref[...] + bv)
    o_ref[1] = jax.nn.relu(s[1] + ra_ref[...] + bv)


def _combine1a(acc, ro, ra, bias):
    bn = 2000
    return pl.pallas_call(
        _combine1a_body,
        grid=(N // bn,),
        in_specs=[
            pl.BlockSpec((NC, 2, bn, H1), lambda i: (0, 0, i, 0)),
            pl.BlockSpec((bn, H1), lambda i: (i, 0)),
            pl.BlockSpec((bn, H1), lambda i: (i, 0)),
            pl.BlockSpec((1, H1), lambda i: (0, 0)),
        ],
        out_specs=pl.BlockSpec((2, bn, H1), lambda i: (0, i, 0)),
        out_shape=jax.ShapeDtypeStruct((2, N, H1), jnp.float32),
    )(acc, ro, ra, bias.reshape(1, H1))


def _combine1b_body(acc_ref, ro_ref, b_ref, o_ref):
    o_ref[...] = jax.nn.relu(acc_ref[0] + acc_ref[1] + ro_ref[...] + b_ref[0])


def _combine1b(acc, ro, bias):
    bn = 2000
    return pl.pallas_call(
        _combine1b_body,
        grid=(N // bn,),
        in_specs=[
            pl.BlockSpec((NC, bn, H1), lambda i: (0, i, 0)),
            pl.BlockSpec((bn, H1), lambda i: (i, 0)),
            pl.BlockSpec((1, H1), lambda i: (0, 0)),
        ],
        out_specs=pl.BlockSpec((bn, H1), lambda i: (i, 0)),
        out_shape=jax.ShapeDtypeStruct((N, H1), jnp.float32),
    )(acc, ro, bias.reshape(1, H1))


def _combine2_body(acc_ref, r0_ref, r1_ref, r2_ref, b_ref, o_ref):
    s = acc_ref[0] + acc_ref[1]
    bv = b_ref[0]
    o_ref[0] = s[0] + r0_ref[...] + bv
    o_ref[1] = s[1] + r1_ref[...] + bv
    o_ref[2] = s[2] + r2_ref[...] + bv


def _combine2(acc, ro3, bias):
    bn = 2000
    return pl.pallas_call(
        _combine2_body,
        grid=(N // bn,),
        in_specs=[
            pl.BlockSpec((NC, 3, bn, H2), lambda i: (0, 0, i, 0)),
            pl.BlockSpec((bn, H2), lambda i: (i, 0)),
            pl.BlockSpec((bn, H2), lambda i: (i, 0)),
            pl.BlockSpec((bn, H2), lambda i: (i, 0)),
            pl.BlockSpec((1, H2), lambda i: (0, 0)),
        ],
        out_specs=pl.BlockSpec((3, bn, H2), lambda i: (0, i, 0)),
        out_shape=jax.ShapeDtypeStruct((3, N, H2), jnp.float32),
    )(acc, ro3[0], ro3[1], ro3[2], bias.reshape(1, H2))


def _postu_body(x2_ref, wb_ref, o_ref):
    h = jax.nn.sigmoid(jnp.mean(x2_ref[...], axis=0))
    u = jnp.dot(wb_ref[0], h[:, None], preferred_element_type=jnp.float32)
    col = lax.broadcasted_iota(jnp.int32, (H2, 128), 1)
    o_ref[...] = jnp.where(col == 0, u, 0.0)


def _post_u(x2_o, wb):
    return pl.pallas_call(
        _postu_body,
        in_specs=[
            pl.BlockSpec((N, H2), lambda: (0, 0)),
            pl.BlockSpec((1, H2, H2), lambda: (0, 0, 0)),
        ],
        out_specs=pl.BlockSpec((H2, 128), lambda: (0, 0)),
        out_shape=jax.ShapeDtypeStruct((H2, 128), jnp.float32),
    )(x2_o, wb)


def _postb_body(x1o_ref, x2_ref, f1_ref, u_ref, att_ref, wq1_ref, wq2_ref,
                wr1_ref, wr2_ref, bb_ref, ros_ref, rosa_ref, q_ref, r_ref):
    bb = bb_ref[0, 0]
    bn = x2_ref.shape[1]
    x2f = x2_ref[...].reshape(3 * bn, H2)
    p = jnp.dot(x2f, u_ref[...], preferred_element_type=jnp.float32)
    bil = (p[:, :1] + bb).reshape(3, bn, 1)
    ros_ref[...] = jnp.concatenate([bil[0], bil[1]], axis=1)
    rosa_ref[...] = jnp.concatenate([bil[0], bil[2]], axis=1)
    fin = jnp.concatenate([att_ref[0, 0] * x1o_ref[...],
                           att_ref[0, 1] * x2_ref[0]], axis=1)
    f1 = f1_ref[...]
    q_ref[...] = (jnp.dot(fin, wq1_ref[...], preferred_element_type=jnp.float32)
                  + jnp.dot(f1, wq2_ref[...], preferred_element_type=jnp.float32))
    r_ref[...] = (jnp.dot(fin, wr1_ref[...], preferred_element_type=jnp.float32)
                  + jnp.dot(f1, wr2_ref[...], preferred_element_type=jnp.float32))


def _post_b(x1_o, x2, features1, u, attt, wq1, wq2, wr1, wr2, bbias):
    bn = 2000
    return pl.pallas_call(
        _postb_body,
        grid=(N // bn,),
        in_specs=[
            pl.BlockSpec((bn, H1), lambda i: (i, 0)),
            pl.BlockSpec((3, bn, H2), lambda i: (0, i, 0)),
            pl.BlockSpec((bn, F_IN), lambda i: (i, 0)),
            pl.BlockSpec((H2, 128), lambda i: (0, 0)),
            pl.BlockSpec((1, 2), lambda i: (0, 0)),
            pl.BlockSpec((H1 + H2, 128), lambda i: (0, 0)),
            pl.BlockSpec((F_IN, 128), lambda i: (0, 0)),
            pl.BlockSpec((H1 + H2, 128), lambda i: (0, 0)),
            pl.BlockSpec((F_IN, 128), lambda i: (0, 0)),
            pl.BlockSpec((1, 1), lambda i: (0, 0)),
        ],
        out_specs=[
            pl.BlockSpec((bn, 2), lambda i: (i, 0)),
            pl.BlockSpec((bn, 2), lambda i: (i, 0)),
            pl.BlockSpec((bn, 128), lambda i: (i, 0)),
            pl.BlockSpec((bn, 128), lambda i: (i, 0)),
        ],
        out_shape=[
            jax.ShapeDtypeStruct((N, 2), jnp.float32),
            jax.ShapeDtypeStruct((N, 2), jnp.float32),
            jax.ShapeDtypeStruct((N, 128), jnp.float32),
            jax.ShapeDtypeStruct((N, 128), jnp.float32),
        ],
    )(x1_o, x2, features1, u, attt, wq1, wq2, wr1, wr2, bbias)


def _logadd_body(g_ref, bc_ref, o_ref):
    s = g_ref[0] + g_ref[1] + bc_ref[0]
    o_ref[...] = s[:, :R]


def _logadd(gathered, bc_pad):
    return pl.pallas_call(
        _logadd_body,
        in_specs=[
            pl.BlockSpec((2, B, 128), lambda: (0, 0, 0)),
            pl.BlockSpec((1, 128), lambda: (0, 0)),
        ],
        out_specs=pl.BlockSpec((B, R), lambda: (0, 0)),
        out_shape=jax.ShapeDtypeStruct((B, R), jnp.float32),
    )(gathered, bc_pad)


def kernel(x_o, x_a, features1, edge_index, edge_type, edge_type1, idx,
           W1, root1, b1, W2, root2, b2, attt, Wb, bbias, Wc, bc):
    src = edge_index[0]
    dst = edge_index[1]
    n32 = jnp.int32(N)
    gidx = (edge_type * n32 + src).reshape(EROWS, KB)
    gidx1 = (edge_type1 * n32 + src).reshape(EROWS, KB)
    seg0 = (edge_type * n32 + dst).reshape(EROWS, KB)
    seg1 = (edge_type1 * n32 + dst).reshape(EROWS, KB)
    dst2 = dst.reshape(EROWS, KB)

    ones_kb = jnp.ones((KB,), jnp.float32)
    zeros_tp = jnp.zeros((CNT_CH,), jnp.float32)
    zeros_h1 = jnp.zeros((ZROWS, H1), jnp.float32)
    zeros_h2 = jnp.zeros((ZROWS, H2), jnp.float32)

    counts = _counts_call(seg0, seg1, ones_kb, zeros_tp)
    inv = _inv_counts(counts.reshape(NC, 2, TPAD))
    inv0 = inv[0]
    inv1 = inv[1]

    w1e = jnp.concatenate([W1, root1[None]], axis=0).astype(jnp.bfloat16)
    w2e = jnp.concatenate([W2, root2[None]], axis=0).astype(jnp.bfloat16)

    t1_o, t1_a = _rel_matmul_multi([x_o.astype(jnp.bfloat16),
                                    x_a.astype(jnp.bfloat16)], w1e)
    ro1_o = t1_o[R * N:]
    ro1_a = t1_a[R * N:]

    acc1a = _agg_l1a(t1_o, t1_a, gidx, gidx1, dst2, seg0, seg1, inv0, inv1,
                     zeros_h1)
    acc1b = _agg_l1b(t1_o, gidx, gidx1, dst2, seg0, seg1, inv0, inv1,
                     zeros_h1)
    x1_01 = _combine1a(acc1a.reshape(NC, 2, NPAD, H1)[:, :, :N],
                       ro1_o, ro1_a, b1)
    x1_b = _combine1b(acc1b.reshape(NC, NPAD, H1)[:, :N], ro1_o, b1)

    x1_bf = x1_01.astype(jnp.bfloat16)
    t2_o, t2_a = _rel_matmul_multi([x1_bf[0], x1_bf[1]], w2e)
    t2_b, = _rel_matmul_multi([x1_b.astype(jnp.bfloat16)], w2e)

    acc2 = _agg_l2(t2_o, t2_a, t2_b, gidx, gidx1, dst2, seg0, seg1, inv0, inv1,
                   zeros_h2)
    x2 = _combine2(acc2.reshape(NC, 3, NPAD, H2)[:, :, :N],
                   (t2_o[R * N:], t2_a[R * N:], t2_b[R * N:]), b2)
    x2_o = x2[0]

    u = _post_u(x2_o, Wb)

    wq1 = jnp.zeros((H1 + H2, 128), jnp.float32).at[:, :R].set(Wc[:, :96].T)
    wq2 = jnp.zeros((F_IN, 128), jnp.float32).at[:, :R].set(Wc[:, 96:224].T)
    wr1 = jnp.zeros((H1 + H2, 128), jnp.float32).at[:, :R].set(Wc[:, 224:320].T)
    wr2 = jnp.zeros((F_IN, 128), jnp.float32).at[:, :R].set(Wc[:, 320:448].T)
    bc_pad = jnp.zeros((1, 128), jnp.float32).at[0, :R].set(bc)

    ret_os, ret_os_a, q, rr = _post_b(
        x1_01[0], x2, features1, u, attt.reshape(1, 2),
        wq1, wq2, wr1, wr2, bbias.reshape(1, 1))

    gathered = _pair_gather_call(q, rr, idx[0], idx[1])
    log = _logadd(gathered, bc_pad)

    return (log, ret_os, ret_os_a, x2_o)


# speedup vs baseline: 4.5251x; 1.3991x over previous
import functools

import jax
import jax.numpy as jnp
from jax import lax
from jax.experimental import pallas as pl
from jax.experimental.pallas import tpu as pltpu
from jax.experimental.pallas import tpu_sc as plsc

N = 10000
E = 320000
R = 65
F_IN = 128
H1 = 64
H2 = 32
B = 4096

RN = R * N
RR = 68
TPAD = 650240
NC = 2
NS = 16
NW = NC * NS
EPW = E // NW
KB = 80
NBLK = EPW // KB
SB = 25
NSB = NBLK // SB
EROWS = E // KB
CNT_SL = TPAD // NS
NPAD = 10240
N_SL = NPAD // NS
BPW = B // NW
CNT_CH = 8128
ZROWS = 64


@functools.cache
def _mesh():
    return plsc.VectorSubcoreMesh(core_axis_name="c", subcore_axis_name="s",
                                  num_cores=NC, num_subcores=NS)


def _wid():
    return lax.axis_index("s") * NC + lax.axis_index("c")


def _sc_counts(seg0_h, seg1_h, ones_h, zeros_h, out_h,
               cnt0_sh, cnt1_sh, ones_v, s0_v, s1_v, zb_v, sem):
    cid = lax.axis_index("c")
    sid = lax.axis_index("s")
    wid = _wid()
    z0 = sid * CNT_SL
    pltpu.sync_copy(zeros_h, zb_v)
    for q in range(CNT_SL // CNT_CH):
        pltpu.sync_copy(zb_v.at[pl.ds(0, CNT_CH)],
                        cnt0_sh.at[pl.ds(z0 + q * CNT_CH, CNT_CH)])
        pltpu.sync_copy(zb_v.at[pl.ds(0, CNT_CH)],
                        cnt1_sh.at[pl.ds(z0 + q * CNT_CH, CNT_CH)])
    pltpu.sync_copy(ones_h, ones_v)
    plsc.subcore_barrier()
    row0 = wid * NBLK
    for s in range(NSB):
        srow = row0 + s * SB
        pltpu.sync_copy(seg0_h.at[pl.ds(srow, SB)], s0_v)
        pltpu.sync_copy(seg1_h.at[pl.ds(srow, SB)], s1_v)

        def blk(jj, carry):
            pltpu.async_copy(ones_v, cnt0_sh.at[s0_v.at[jj]], sem, add=True)
            pltpu.async_copy(ones_v, cnt1_sh.at[s1_v.at[jj]], sem, add=True)
            return carry

        lax.fori_loop(0, SB, blk, 0)

        def drain(jj, carry):
            pltpu.make_async_copy(ones_v, cnt0_sh.at[s0_v.at[jj]], sem).wait()
            pltpu.make_async_copy(ones_v, cnt1_sh.at[s1_v.at[jj]], sem).wait()
            return carry

        lax.fori_loop(0, SB, drain, 0)
    plsc.subcore_barrier()
    for t, sh in ((0, cnt0_sh), (1, cnt1_sh)):
        for q in range(CNT_SL // CNT_CH):
            pltpu.sync_copy(sh.at[pl.ds(z0 + q * CNT_CH, CNT_CH)], zb_v)
            pltpu.sync_copy(
                zb_v,
                out_h.at[pl.ds((cid * 2 + t) * TPAD + z0 + q * CNT_CH, CNT_CH)])


@functools.cache
def _counts_kernel():
    return pl.kernel(
        _sc_counts,
        out_type=jax.ShapeDtypeStruct((NC * 2 * TPAD,), jnp.float32),
        mesh=_mesh(),
        compiler_params=pltpu.CompilerParams(use_tc_tiling_on_sc=False),
        scratch_types=[
            pltpu.VMEM_SHARED((TPAD,), jnp.float32),
            pltpu.VMEM_SHARED((TPAD,), jnp.float32),
            pltpu.VMEM((KB,), jnp.float32),
            pltpu.VMEM((SB, KB), jnp.int32),
            pltpu.VMEM((SB, KB), jnp.int32),
            pltpu.VMEM((CNT_CH,), jnp.float32),
            pltpu.SemaphoreType.DMA,
        ],
    )


def _counts_call(*args):
    return _counts_kernel()(*args)


def _scale_rows(msg_ref, ew_ref, h):
    nh = h // 16

    def grp(g, carry):
        w16 = ew_ref[pl.ds(g * 16, 16)]
        for e in range(16):
            ei = g * 16 + e
            w = w16[e]
            for k in range(nh):
                sl = pl.ds(k * 16, 16)
                msg_ref[ei, sl] = msg_ref[ei, sl] * w
        return carry

    lax.fori_loop(0, KB // 16, grp, 0)


def _make_agg(h, n_tables, branches):
    nb = len(branches)
    any_main = any(not alt for _, alt in branches)
    any_alt = any(alt for _, alt in branches)

    def body(*refs):
        tabs = refs[:n_tables]
        it = iter(refs[n_tables:])
        gidx_h, gidx1_h, dst_h, seg0_h, seg1_h = (next(it) for _ in range(5))
        inv0_h, inv1_h, zeros_h, out_h = (next(it) for _ in range(4))
        accs = [next(it) for _ in range(nb)]
        msgs = [[next(it), next(it)] for _ in range(nb)]
        g_v, g1_v, dst_v, s0_v, s1_v, zb_v = (next(it) for _ in range(6))
        ew0_v = [next(it), next(it)]
        ew1_v = [next(it), next(it)]
        sems = [[next(it), next(it)] for _ in range(nb)]
        seme0 = [next(it), next(it)]
        seme1 = [next(it), next(it)]
        cid = lax.axis_index("c")
        sid = lax.axis_index("s")
        wid = _wid()
        r0 = sid * N_SL
        pltpu.sync_copy(zeros_h, zb_v)
        for acc in accs:
            for q in range(N_SL // ZROWS):
                pltpu.sync_copy(zb_v, acc.at[pl.ds(r0 + q * ZROWS, ZROWS)])
        plsc.subcore_barrier()
        row0 = wid * NBLK
        gsel = [g1_v if alt else g_v for _, alt in branches]
        esel = [ew1_v if alt else ew0_v for _, alt in branches]

        def fire(jj, par):
            if any_main:
                pltpu.async_copy(inv0_h.at[s0_v.at[jj]], ew0_v[par],
                                 seme0[par])
            if any_alt:
                pltpu.async_copy(inv1_h.at[s1_v.at[jj]], ew1_v[par],
                                 seme1[par])
            for b, (slot, _) in enumerate(branches):
                pltpu.async_copy(tabs[slot].at[gsel[b].at[jj]],
                                 msgs[b][par], sems[b][par])

        def process(jj, par):
            if any_main:
                pltpu.make_async_copy(inv0_h.at[s0_v.at[jj]], ew0_v[par],
                                      seme0[par]).wait()
            if any_alt:
                pltpu.make_async_copy(inv1_h.at[s1_v.at[jj]], ew1_v[par],
                                      seme1[par]).wait()
            for b, (slot, _) in enumerate(branches):
                pltpu.make_async_copy(tabs[slot].at[gsel[b].at[jj]],
                                      msgs[b][par], sems[b][par]).wait()
                _scale_rows(msgs[b][par],
                            ew1_v[par] if branches[b][1] else ew0_v[par], h)
                pltpu.sync_copy(msgs[b][par], accs[b].at[dst_v.at[jj]],
                                add=True)

        for s in range(NSB):
            srow = row0 + s * SB
            if any_main:
                pltpu.sync_copy(gidx_h.at[pl.ds(srow, SB)], g_v)
                pltpu.sync_copy(seg0_h.at[pl.ds(srow, SB)], s0_v)
            if any_alt:
                pltpu.sync_copy(gidx1_h.at[pl.ds(srow, SB)], g1_v)
                pltpu.sync_copy(seg1_h.at[pl.ds(srow, SB)], s1_v)
            pltpu.sync_copy(dst_h.at[pl.ds(srow, SB)], dst_v)
            fire(0, 0)

            def pair(t, carry):
                j0 = 2 * t
                fire(j0 + 1, 1)
                process(j0, 0)
                fire(j0 + 2, 0)
                process(j0 + 1, 1)
                return carry

            lax.fori_loop(0, (SB - 1) // 2, pair, 0)
            process(SB - 1, 0)
        plsc.subcore_barrier()
        for b in range(nb):
            for q in range(N_SL // ZROWS):
                pltpu.sync_copy(accs[b].at[pl.ds(r0 + q * ZROWS, ZROWS)], zb_v)
                pltpu.sync_copy(
                    zb_v,
                    out_h.at[pl.ds((cid * nb + b) * NPAD + r0 + q * ZROWS,
                                   ZROWS)])

    scratch = (
        [pltpu.VMEM_SHARED((NPAD, h), jnp.float32)] * nb
        + [pltpu.VMEM((KB, h), jnp.float32)] * (2 * nb)
        + [pltpu.VMEM((SB, KB), jnp.int32)] * 5
        + [pltpu.VMEM((ZROWS, h), jnp.float32)]
        + [pltpu.VMEM((KB,), jnp.float32)] * 4
        + [pltpu.SemaphoreType.DMA] * (2 * nb + 4)
    )
    return pl.kernel(
        body,
        out_type=jax.ShapeDtypeStruct((NC * nb * NPAD, h), jnp.float32),
        mesh=_mesh(),
        compiler_params=pltpu.CompilerParams(use_tc_tiling_on_sc=False),
        scratch_types=scratch,
    )


_make_agg = functools.cache(_make_agg)


def _agg_l1a(*args):
    return _make_agg(H1, 2, ((0, False), (1, False)))(*args)


def _agg_l1b(*args):
    return _make_agg(H1, 1, ((0, True),))(*args)


def _agg_l2(*args):
    return _make_agg(H2, 3, ((0, False), (1, False), (2, True)))(*args)


def _sc_pair_gather(q_h, r_h, aa_h, bb_h, out_h, i_v, rows_v, sem):
    wid = _wid()
    base = wid * BPW
    pltpu.sync_copy(aa_h.at[pl.ds(base, BPW)], i_v)
    pltpu.async_copy(q_h.at[i_v], rows_v, sem).wait()
    pltpu.sync_copy(rows_v, out_h.at[0, pl.ds(base, BPW)])
    pltpu.sync_copy(bb_h.at[pl.ds(base, BPW)], i_v)
    pltpu.async_copy(r_h.at[i_v], rows_v, sem).wait()
    pltpu.sync_copy(rows_v, out_h.at[1, pl.ds(base, BPW)])


@functools.cache
def _pair_gather_kernel():
    return pl.kernel(
        _sc_pair_gather,
        out_type=jax.ShapeDtypeStruct((2, B, 128), jnp.float32),
        mesh=_mesh(),
        scratch_types=[
            pltpu.VMEM((BPW,), jnp.int32),
            pltpu.VMEM((BPW, 128), jnp.float32),
            pltpu.SemaphoreType.DMA,
        ],
    )


def _pair_gather_call(*args):
    return _pair_gather_kernel()(*args)


def _make_relmm_body(m):
    def body(*refs):
        wb = refs[m][...]
        for i in range(m):
            refs[m + 1 + i][...] = jnp.dot(refs[i][...], wb,
                                           preferred_element_type=jnp.float32)
    return body


def _rel_matmul_multi(xs, w_flat, h):
    m = len(xs)
    n, f = xs[0].shape
    cols = w_flat.shape[1]
    bn = 1000
    ncb = cols // 2176
    cb = cols // ncb
    outs = pl.pallas_call(
        _make_relmm_body(m),
        grid=(n // bn, ncb),
        in_specs=[pl.BlockSpec((bn, f), lambda i, j: (i, 0))
                  for _ in range(m)]
        + [pl.BlockSpec((f, cb), lambda i, j: (0, j))],
        out_specs=[pl.BlockSpec((bn, cb), lambda i, j: (i, j))
                   for _ in range(m)],
        out_shape=[jax.ShapeDtypeStruct((n, cols), jnp.float32)
                   for _ in range(m)],
    )(*xs, w_flat)
    return [o.reshape(n * RR, h) for o in outs]


def _inv_body(c_ref, o_ref):
    s = c_ref[0] + c_ref[1]
    o_ref[...] = 1.0 / jnp.maximum(s, 1.0)


def _inv_counts(counts):
    c4 = counts.reshape(NC, 2, TPAD // 128, 128)
    nb = TPAD // 128
    bn = 1016
    out = pl.pallas_call(
        _inv_body,
        grid=(nb // bn,),
        in_specs=[pl.BlockSpec((NC, 2, bn, 128), lambda i: (0, 0, i, 0))],
        out_specs=pl.BlockSpec((2, bn, 128), lambda i: (0, i, 0)),
        out_shape=jax.ShapeDtypeStruct((2, nb, 128), jnp.float32),
    )(c4)
    return out.reshape(2, TPAD)


def _combine1a_body(acc_ref, ro_ref, ra_ref, b_ref, o_ref):
    s = acc_ref[0] + acc_ref[1]
    bv = b_ref[0]
    o_ref[0] = jax.nn.relu(s[0] + ro_ref[...] + bv)
    o_ref[1] = jax.nn.relu(s[1] + ra_ref[...] + bv)


def _combine1a(acc, ro, ra, bias):
    bn = 2000
    return pl.pallas_call(
        _combine1a_body,
        grid=(N // bn,),
        in_specs=[
            pl.BlockSpec((NC, 2, bn, H1), lambda i: (0, 0, i, 0)),
            pl.BlockSpec((bn, H1), lambda i: (i, 0)),
            pl.BlockSpec((bn, H1), lambda i: (i, 0)),
            pl.BlockSpec((1, H1), lambda i: (0, 0)),
        ],
        out_specs=pl.BlockSpec((2, bn, H1), lambda i: (0, i, 0)),
        out_shape=jax.ShapeDtypeStruct((2, N, H1), jnp.float32),
    )(acc, ro, ra, bias.reshape(1, H1))


def _combine1b_body(acc_ref, ro_ref, b_ref, o_ref):
    o_ref[...] = jax.nn.relu(acc_ref[0] + acc_ref[1] + ro_ref[...] + b_ref[0])


def _combine1b(acc, ro, bias):
    bn = 2000
    return pl.pallas_call(
        _combine1b_body,
        grid=(N // bn,),
        in_specs=[
            pl.BlockSpec((NC, bn, H1), lambda i: (0, i, 0)),
            pl.BlockSpec((bn, H1), lambda i: (i, 0)),
            pl.BlockSpec((1, H1), lambda i: (0, 0)),
        ],
        out_specs=pl.BlockSpec((bn, H1), lambda i: (i, 0)),
        out_shape=jax.ShapeDtypeStruct((N, H1), jnp.float32),
    )(acc, ro, bias.reshape(1, H1))


def _combine2_body(acc_ref, r0_ref, r1_ref, r2_ref, b_ref, o_ref):
    s = acc_ref[0] + acc_ref[1]
    bv = b_ref[0]
    o_ref[0] = s[0] + r0_ref[...] + bv
    o_ref[1] = s[1] + r1_ref[...] + bv
    o_ref[2] = s[2] + r2_ref[...] + bv


def _combine2(acc, ro3, bias):
    bn = 2000
    return pl.pallas_call(
        _combine2_body,
        grid=(N // bn,),
        in_specs=[
            pl.BlockSpec((NC, 3, bn, H2), lambda i: (0, 0, i, 0)),
            pl.BlockSpec((bn, H2), lambda i: (i, 0)),
            pl.BlockSpec((bn, H2), lambda i: (i, 0)),
            pl.BlockSpec((bn, H2), lambda i: (i, 0)),
            pl.BlockSpec((1, H2), lambda i: (0, 0)),
        ],
        out_specs=pl.BlockSpec((3, bn, H2), lambda i: (0, i, 0)),
        out_shape=jax.ShapeDtypeStruct((3, N, H2), jnp.float32),
    )(acc, ro3[0], ro3[1], ro3[2], bias.reshape(1, H2))


def _postu_body(x2_ref, wb_ref, o_ref):
    h = jax.nn.sigmoid(jnp.mean(x2_ref[...], axis=0))
    u = jnp.dot(wb_ref[0], h[:, None], preferred_element_type=jnp.float32)
    col = lax.broadcasted_iota(jnp.int32, (H2, 128), 1)
    o_ref[...] = jnp.where(col == 0, u, 0.0)


def _post_u(x2_o, wb):
    return pl.pallas_call(
        _postu_body,
        in_specs=[
            pl.BlockSpec((N, H2), lambda: (0, 0)),
            pl.BlockSpec((1, H2, H2), lambda: (0, 0, 0)),
        ],
        out_specs=pl.BlockSpec((H2, 128), lambda: (0, 0)),
        out_shape=jax.ShapeDtypeStruct((H2, 128), jnp.float32),
    )(x2_o, wb)


def _postb_body(x1o_ref, x2_ref, f1_ref, u_ref, att_ref, wq1_ref, wq2_ref,
                wr1_ref, wr2_ref, bb_ref, ros_ref, rosa_ref, q_ref, r_ref):
    bb = bb_ref[0, 0]
    bn = x2_ref.shape[1]
    x2f = x2_ref[...].reshape(3 * bn, H2)
    p = jnp.dot(x2f, u_ref[...], preferred_element_type=jnp.float32)
    bil = (p[:, :1] + bb).reshape(3, bn, 1)
    ros_ref[...] = jnp.concatenate([bil[0], bil[1]], axis=1)
    rosa_ref[...] = jnp.concatenate([bil[0], bil[2]], axis=1)
    fin = jnp.concatenate([att_ref[0, 0] * x1o_ref[...],
                           att_ref[0, 1] * x2_ref[0]], axis=1)
    f1 = f1_ref[...]
    q_ref[...] = (jnp.dot(fin, wq1_ref[...], preferred_element_type=jnp.float32)
                  + jnp.dot(f1, wq2_ref[...], preferred_element_type=jnp.float32))
    r_ref[...] = (jnp.dot(fin, wr1_ref[...], preferred_element_type=jnp.float32)
                  + jnp.dot(f1, wr2_ref[...], preferred_element_type=jnp.float32))


def _post_b(x1_o, x2, features1, u, attt, wq1, wq2, wr1, wr2, bbias):
    bn = 2000
    return pl.pallas_call(
        _postb_body,
        grid=(N // bn,),
        in_specs=[
            pl.BlockSpec((bn, H1), lambda i: (i, 0)),
            pl.BlockSpec((3, bn, H2), lambda i: (0, i, 0)),
            pl.BlockSpec((bn, F_IN), lambda i: (i, 0)),
            pl.BlockSpec((H2, 128), lambda i: (0, 0)),
            pl.BlockSpec((1, 2), lambda i: (0, 0)),
            pl.BlockSpec((H1 + H2, 128), lambda i: (0, 0)),
            pl.BlockSpec((F_IN, 128), lambda i: (0, 0)),
            pl.BlockSpec((H1 + H2, 128), lambda i: (0, 0)),
            pl.BlockSpec((F_IN, 128), lambda i: (0, 0)),
            pl.BlockSpec((1, 1), lambda i: (0, 0)),
        ],
        out_specs=[
            pl.BlockSpec((bn, 2), lambda i: (i, 0)),
            pl.BlockSpec((bn, 2), lambda i: (i, 0)),
            pl.BlockSpec((bn, 128), lambda i: (i, 0)),
            pl.BlockSpec((bn, 128), lambda i: (i, 0)),
        ],
        out_shape=[
            jax.ShapeDtypeStruct((N, 2), jnp.float32),
            jax.ShapeDtypeStruct((N, 2), jnp.float32),
            jax.ShapeDtypeStruct((N, 128), jnp.float32),
            jax.ShapeDtypeStruct((N, 128), jnp.float32),
        ],
    )(x1_o, x2, features1, u, attt, wq1, wq2, wr1, wr2, bbias)


def _logadd_body(g_ref, bc_ref, o_ref):
    s = g_ref[0] + g_ref[1] + bc_ref[0]
    o_ref[...] = s[:, :R]


def _logadd(gathered, bc_pad):
    return pl.pallas_call(
        _logadd_body,
        in_specs=[
            pl.BlockSpec((2, B, 128), lambda: (0, 0, 0)),
            pl.BlockSpec((1, 128), lambda: (0, 0)),
        ],
        out_specs=pl.BlockSpec((B, R), lambda: (0, 0)),
        out_shape=jax.ShapeDtypeStruct((B, R), jnp.float32),
    )(gathered, bc_pad)


def kernel(x_o, x_a, features1, edge_index, edge_type, edge_type1, idx,
           W1, root1, b1, W2, root2, b2, attt, Wb, bbias, Wc, bc):
    src = edge_index[0]
    dst = edge_index[1]
    n32 = jnp.int32(N)
    rr32 = jnp.int32(RR)
    gidx = (src * rr32 + edge_type).reshape(EROWS, KB)
    gidx1 = (src * rr32 + edge_type1).reshape(EROWS, KB)
    seg0 = (edge_type * n32 + dst).reshape(EROWS, KB)
    seg1 = (edge_type1 * n32 + dst).reshape(EROWS, KB)
    dst2 = dst.reshape(EROWS, KB)

    ones_kb = jnp.ones((KB,), jnp.float32)
    zeros_tp = jnp.zeros((CNT_CH,), jnp.float32)
    zeros_h1 = jnp.zeros((ZROWS, H1), jnp.float32)
    zeros_h2 = jnp.zeros((ZROWS, H2), jnp.float32)

    counts = _counts_call(seg0, seg1, ones_kb, zeros_tp)
    inv = _inv_counts(counts.reshape(NC, 2, TPAD))
    inv0 = inv[0]
    inv1 = inv[1]

    w1e = jnp.concatenate(
        [W1, root1[None], jnp.zeros((RR - R - 1, F_IN, H1), jnp.float32)],
        axis=0).transpose(1, 0, 2).reshape(F_IN, RR * H1).astype(jnp.bfloat16)
    w2e = jnp.concatenate(
        [W2, root2[None], jnp.zeros((RR - R - 1, H1, H2), jnp.float32)],
        axis=0).transpose(1, 0, 2).reshape(H1, RR * H2).astype(jnp.bfloat16)

    t1_o, t1_a = _rel_matmul_multi([x_o.astype(jnp.bfloat16),
                                    x_a.astype(jnp.bfloat16)], w1e, H1)
    ro1_o = t1_o.reshape(N, RR, H1)[:, R]
    ro1_a = t1_a.reshape(N, RR, H1)[:, R]

    acc1a = _agg_l1a(t1_o, t1_a, gidx, gidx1, dst2, seg0, seg1, inv0, inv1,
                     zeros_h1)
    acc1b = _agg_l1b(t1_o, gidx, gidx1, dst2, seg0, seg1, inv0, inv1,
                     zeros_h1)
    x1_01 = _combine1a(acc1a.reshape(NC, 2, NPAD, H1)[:, :, :N],
                       ro1_o, ro1_a, b1)
    x1_b = _combine1b(acc1b.reshape(NC, NPAD, H1)[:, :N], ro1_o, b1)

    x1_bf = x1_01.astype(jnp.bfloat16)
    t2_o, t2_a = _rel_matmul_multi([x1_bf[0], x1_bf[1]], w2e, H2)
    t2_b, = _rel_matmul_multi([x1_b.astype(jnp.bfloat16)], w2e, H2)

    acc2 = _agg_l2(t2_o, t2_a, t2_b, gidx, gidx1, dst2, seg0, seg1, inv0, inv1,
                   zeros_h2)
    x2 = _combine2(acc2.reshape(NC, 3, NPAD, H2)[:, :, :N],
                   (t2_o.reshape(N, RR, H2)[:, R],
                    t2_a.reshape(N, RR, H2)[:, R],
                    t2_b.reshape(N, RR, H2)[:, R]), b2)
    x2_o = x2[0]

    u = _post_u(x2_o, Wb)

    wq1 = jnp.zeros((H1 + H2, 128), jnp.float32).at[:, :R].set(Wc[:, :96].T)
    wq2 = jnp.zeros((F_IN, 128), jnp.float32).at[:, :R].set(Wc[:, 96:224].T)
    wr1 = jnp.zeros((H1 + H2, 128), jnp.float32).at[:, :R].set(Wc[:, 224:320].T)
    wr2 = jnp.zeros((F_IN, 128), jnp.float32).at[:, :R].set(Wc[:, 320:448].T)
    bc_pad = jnp.zeros((1, 128), jnp.float32).at[0, :R].set(bc)

    ret_os, ret_os_a, q, rr = _post_b(
        x1_01[0], x2, features1, u, attt.reshape(1, 2),
        wq1, wq2, wr1, wr2, bbias.reshape(1, 1))

    gathered = _pair_gather_call(q, rr, idx[0], idx[1])
    log = _logadd(gathered, bc_pad)

    return (log, ret_os, ret_os_a, x2_o)
